# Initial kernel scaffold; baseline (speedup 1.0000x reference)
import functools

import jax
import jax.numpy as jnp
from jax import lax
from jax.experimental import pallas as pl
from jax.experimental.pallas import tpu as pltpu
from jax.experimental.pallas import tpu_sc as plsc

_BATCH = 4096
_SLOT_NUM = 26
_NNZ = 4
_EMB = 32

_B = _BATCH * _SLOT_NUM * _NNZ
_NW = 32
_BPW = _B // _NW
_CHUNK = 128
_NCH = _BPW // _CHUNK

_mesh = plsc.VectorSubcoreMesh(core_axis_name="c", subcore_axis_name="s")


@functools.partial(
    pl.kernel,
    mesh=_mesh,
    out_type=jax.ShapeDtypeStruct((_B, _EMB), jnp.float32),
    scratch_types=[
        pltpu.VMEM((_BPW,), jnp.int32),
        pltpu.VMEM((2, _CHUNK, _EMB), jnp.float32),
        pltpu.SemaphoreType.DMA,
    ],
)
def _sc_gather(idx_hbm, table_hbm, out_hbm, idx_v, rows_v, sem_g):
    wid = lax.axis_index("s") * 2 + lax.axis_index("c")
    base = wid * _BPW

    pltpu.sync_copy(idx_hbm.at[pl.ds(base, _BPW)], idx_v)

    def gather_start(i, buf):
        return pltpu.async_copy(
            table_hbm.at[idx_v.at[pl.ds(i * _CHUNK, _CHUNK)]],
            rows_v.at[buf],
            sem_g,
        )

    gather_start(0, 0).wait()

    def body(i, _):
        buf = lax.rem(i, 2)
        cp = gather_start(i + 1, 1 - buf)
        pltpu.sync_copy(
            rows_v.at[buf], out_hbm.at[pl.ds(base + i * _CHUNK, _CHUNK)]
        )
        cp.wait()
        return 0

    lax.fori_loop(0, _NCH - 1, body, 0, unroll=False)

    buf = (_NCH - 1) % 2
    pltpu.sync_copy(
        rows_v.at[buf], out_hbm.at[pl.ds(base + (_NCH - 1) * _CHUNK, _CHUNK)]
    )


def kernel(inputs, table):
    flat = inputs.reshape(-1).astype(jnp.int32)
    out = _sc_gather(flat, table)
    return out.reshape(_BATCH, _SLOT_NUM, _NNZ, _EMB)


# speedup vs baseline: 1.0207x; 1.0207x over previous
import functools

import jax
import jax.numpy as jnp
from jax import lax
from jax.experimental import pallas as pl
from jax.experimental.pallas import tpu as pltpu
from jax.experimental.pallas import tpu_sc as plsc

_BATCH = 4096
_SLOT_NUM = 26
_NNZ = 4
_EMB = 32

_B = _BATCH * _SLOT_NUM * _NNZ
_NW = 32
_BPW = _B // _NW
_CHUNK = 128
_NCH = _BPW // _CHUNK

_mesh = plsc.VectorSubcoreMesh(core_axis_name="c", subcore_axis_name="s")


@functools.partial(
    pl.kernel,
    mesh=_mesh,
    out_type=jax.ShapeDtypeStruct((_B, _EMB), jnp.float32),
    scratch_types=[
        pltpu.VMEM((_BPW,), jnp.int32),
        pltpu.VMEM((2, _CHUNK, _EMB), jnp.float32),
        pltpu.SemaphoreType.DMA,
    ],
    compiler_params=pltpu.CompilerParams(use_tc_tiling_on_sc=False),
)
def _sc_gather(idx_hbm, table_hbm, out_hbm, idx_v, rows_v, sem_g):
    wid = lax.axis_index("s") * 2 + lax.axis_index("c")
    base = wid * _BPW

    pltpu.sync_copy(idx_hbm.at[pl.ds(base, _BPW)], idx_v)

    def gather_start(i, buf):
        return pltpu.async_copy(
            table_hbm.at[idx_v.at[pl.ds(i * _CHUNK, _CHUNK)]],
            rows_v.at[buf],
            sem_g,
        )

    gather_start(0, 0).wait()

    def body(i, _):
        buf = lax.rem(i, 2)
        cp = gather_start(i + 1, 1 - buf)
        pltpu.sync_copy(
            rows_v.at[buf], out_hbm.at[pl.ds(base + i * _CHUNK, _CHUNK)]
        )
        cp.wait()
        return 0

    lax.fori_loop(0, _NCH - 1, body, 0, unroll=False)

    buf = (_NCH - 1) % 2
    pltpu.sync_copy(
        rows_v.at[buf], out_hbm.at[pl.ds(base + (_NCH - 1) * _CHUNK, _CHUNK)]
    )


def kernel(inputs, table):
    flat = inputs.reshape(-1).astype(jnp.int32)
    out = _sc_gather(flat, table)
    return out.reshape(_BATCH, _SLOT_NUM, _NNZ, _EMB)
